# baseline (speedup 1.0000x reference)
import functools

import jax
import jax.numpy as jnp
from jax.experimental import pallas as pl
from jax.experimental.pallas import tpu as pltpu
from jax.experimental.pallas import tpu_sc as plsc

_W = 256
_Q = 4


def _sc_gather(table_hbm_arr, idx_arr, m):
    mesh = plsc.VectorSubcoreMesh(core_axis_name="c", subcore_axis_name="s")

    @functools.partial(
        pl.kernel,
        out_type=jax.ShapeDtypeStruct((m, 128), jnp.float32),
        mesh=mesh,
    )
    def gather_kernel(table_hbm, idx_hbm, out_hbm):
        def body(i_vmem, o_vmem):
            pltpu.sync_copy(table_hbm.at[i_vmem.at[0]], o_vmem)

        pltpu.emit_pipeline(
            body,
            grid=(m // _W,),
            in_specs=[pl.BlockSpec((1, _W), lambda i: (0, i))],
            out_specs=[pl.BlockSpec((_W, 128), lambda i: (i, 0))],
            core_axis_name=("c", "s"),
            dimension_semantics=(pltpu.PARALLEL,),
        )(idx_hbm, out_hbm)

    return gather_kernel(table_hbm_arr, idx_arr.reshape(1, m))


def _tc_compact(wide, out_prev, q, batch, seq, dim):
    bq = batch // _Q
    rb = 64
    wide3 = wide.reshape(bq, seq, 128)

    kwargs = {}
    operands = [wide3]
    in_specs = [pl.BlockSpec((rb, seq, 128), lambda i: (i, 0, 0))]
    if out_prev is not None:
        operands = [out_prev, wide3]
        in_specs = [pl.BlockSpec(memory_space=pl.ANY)] + in_specs
        kwargs["input_output_aliases"] = {0: 0}

        def body(prev_ref, w_ref, o_ref):
            del prev_ref
            o_ref[...] = w_ref[:, :, :dim]
    else:
        def body(w_ref, o_ref):
            o_ref[...] = w_ref[:, :, :dim]

    return pl.pallas_call(
        body,
        out_shape=jax.ShapeDtypeStruct((batch, seq, dim), jnp.float32),
        grid=(bq // rb,),
        in_specs=in_specs,
        out_specs=pl.BlockSpec(
            (rb, seq, dim), lambda i, _q=q: (_q * (bq // rb) + i, 0, 0)
        ),
        **kwargs,
    )(*operands)


def kernel(token_ids, embed_matrix):
    batch, seq = token_ids.shape
    _, dim = embed_matrix.shape
    n = batch * seq
    idx = token_ids.reshape(n).astype(jnp.int32)
    table = jnp.pad(embed_matrix, ((0, 0), (0, 128 - dim)))

    m = n // _Q
    out = None
    for q in range(_Q):
        wide = _sc_gather(table, idx[q * m:(q + 1) * m], m)
        out = _tc_compact(wide, out, q, batch, seq, dim)
    return out


# speedup vs baseline: 1.3595x; 1.3595x over previous
import functools

import jax
import jax.numpy as jnp
from jax.experimental import pallas as pl
from jax.experimental.pallas import tpu as pltpu
from jax.experimental.pallas import tpu_sc as plsc

_W = 256


def kernel(token_ids, embed_matrix):
    batch, seq = token_ids.shape
    _, dim = embed_matrix.shape
    n = batch * seq
    idx = token_ids.reshape(1, n).astype(jnp.int32)
    table = jnp.pad(embed_matrix, ((0, 0), (0, 128 - dim)))

    mesh = plsc.VectorSubcoreMesh(core_axis_name="c", subcore_axis_name="s")

    @functools.partial(
        pl.kernel,
        out_type=jax.ShapeDtypeStruct((n, 128), embed_matrix.dtype),
        mesh=mesh,
    )
    def gather_kernel(table_hbm, idx_hbm, out_hbm):
        def body(i_vmem, o_vmem):
            pltpu.sync_copy(table_hbm.at[i_vmem.at[0]], o_vmem)

        pltpu.emit_pipeline(
            body,
            grid=(n // _W,),
            in_specs=[pl.BlockSpec((1, _W), lambda i: (0, i))],
            out_specs=[pl.BlockSpec((_W, 128), lambda i: (i, 0))],
            core_axis_name=("c", "s"),
            dimension_semantics=(pltpu.PARALLEL,),
        )(idx_hbm, out_hbm)

    out = gather_kernel(table, idx)
    return out.reshape(batch, seq, 128)[:, :, :dim]
